# Initial kernel scaffold; baseline (speedup 1.0000x reference)
import functools

import jax
import jax.numpy as jnp
from jax import lax
from jax.experimental import pallas as pl
from jax.experimental.pallas import tpu as pltpu
from jax.experimental.pallas import tpu_sc as plsc

B, L, NL = 4096, 200, 4
D = 64
N = B * L
NC, NS = 2, 16
NW = NC * NS
NPW = N // NW
C = 128
NCH = NPW // C

_mesh = plsc.VectorSubcoreMesh(core_axis_name="c", subcore_axis_name="s")


@functools.partial(
    pl.kernel,
    out_type=jax.ShapeDtypeStruct((N, D), jnp.float32),
    mesh=_mesh,
    scratch_types=[
        pltpu.VMEM((NL, C), jnp.int32),
        pltpu.VMEM((NL, C, D), jnp.float32),
        pltpu.SemaphoreType.DMA,
    ],
)
def _poi_gather(idx_hbm, w0, w1, w2, w3, out_hbm, idx_v, rows_v, sem):
    wid = lax.axis_index("s") * NC + lax.axis_index("c")
    tables = (w0, w1, w2, w3)

    def chunk(g, carry):
        base = wid * NPW + g * C
        pltpu.sync_copy(idx_hbm.at[wid, g], idx_v)
        cps = [
            pltpu.async_copy(tables[k].at[idx_v.at[k]], rows_v.at[k], sem)
            for k in range(NL)
        ]
        for cp in cps:
            cp.wait()

        def row(r, carry2):
            for c4 in range(D // 16):
                s = pl.ds(c4 * 16, 16)
                v = ((rows_v[0, r, s] + rows_v[1, r, s])
                     + rows_v[2, r, s]) + rows_v[3, r, s]
                rows_v[0, r, s] = v * 0.25
            return carry2

        lax.fori_loop(0, C, row, 0)
        pltpu.sync_copy(rows_v.at[0], out_hbm.at[pl.ds(base, C), :])
        return carry

    lax.fori_loop(0, NCH, chunk, 0)


def kernel(poi_path, W0, W1, W2, W3):
    idx = poi_path.astype(jnp.int32).reshape(NW, NCH, C, NL)
    idx = idx.transpose(0, 1, 3, 2)
    out = _poi_gather(idx, W0, W1, W2, W3)
    return out.reshape(B, L, D)


# speedup vs baseline: 7.3370x; 7.3370x over previous
import functools

import jax
import jax.numpy as jnp
from jax import lax
from jax.experimental import pallas as pl
from jax.experimental.pallas import tpu as pltpu
from jax.experimental.pallas import tpu_sc as plsc

B, L, NL = 4096, 200, 4
D = 64
N = B * L
NC, NS = 2, 16
NW = NC * NS
NPW = N // NW
C = 128
NCH = NPW // C

_mesh = plsc.VectorSubcoreMesh(core_axis_name="c", subcore_axis_name="s")


@functools.partial(
    pl.kernel,
    out_type=jax.ShapeDtypeStruct((N, D), jnp.float32),
    mesh=_mesh,
    scratch_types=[
        pltpu.VMEM((NL, C), jnp.int32),
        pltpu.VMEM((NL, C, D), jnp.float32),
        pltpu.SemaphoreType.DMA,
    ],
    compiler_params=pltpu.CompilerParams(use_tc_tiling_on_sc=False),
)
def _poi_gather(idx_hbm, w0, w1, w2, w3, out_hbm, idx_v, rows_v, sem):
    wid = lax.axis_index("s") * NC + lax.axis_index("c")
    tables = (w0, w1, w2, w3)

    def chunk(g, carry):
        base = wid * NPW + g * C
        pltpu.sync_copy(idx_hbm.at[wid, g], idx_v)
        cps = [
            pltpu.async_copy(tables[k].at[idx_v.at[k]], rows_v.at[k], sem)
            for k in range(NL)
        ]
        for cp in cps:
            cp.wait()

        def row(r, carry2):
            for c4 in range(D // 16):
                s = pl.ds(c4 * 16, 16)
                v = ((rows_v[0, r, s] + rows_v[1, r, s])
                     + rows_v[2, r, s]) + rows_v[3, r, s]
                rows_v[0, r, s] = v * 0.25
            return carry2

        lax.fori_loop(0, C, row, 0)
        pltpu.sync_copy(rows_v.at[0], out_hbm.at[pl.ds(base, C), :])
        return carry

    lax.fori_loop(0, NCH, chunk, 0)


def kernel(poi_path, W0, W1, W2, W3):
    idx = poi_path.astype(jnp.int32).reshape(NW, NCH, C, NL)
    idx = idx.transpose(0, 1, 3, 2)
    out = _poi_gather(idx, W0, W1, W2, W3)
    return out.reshape(B, L, D)
